# Initial kernel scaffold; baseline (speedup 1.0000x reference)
import jax
import jax.numpy as jnp
from jax.experimental import pallas as pl


def kernel(adj, adj_orig, features, W_base, W_mean, W_nc0, b_nc0, W_nc1, b_nc1):
    raise NotImplementedError("write your pallas kernel here")


# speedup vs baseline: 5.9148x; 5.9148x over previous
import jax
import jax.numpy as jnp
from jax.experimental import pallas as pl

_N = 4096
_TM = 512
_NT = _N // _TM


def _feats_kernel(features_ref, wb_ref, wn0_ref, fb_ref, f2_ref):
    f = features_ref[...]
    fb_ref[...] = jnp.dot(f, wb_ref[...])
    f2_ref[...] = jnp.dot(f, wn0_ref[...])


def _hidden_kernel(adj_ref, fb_ref, wm_ref, t2_ref):
    t2_ref[...] = jnp.dot(jnp.dot(adj_ref[...], fb_ref[...]), wm_ref[...])


def _mean_kernel(adj_ref, t2_ref, mean_ref):
    mean_ref[...] = jax.nn.relu(jnp.dot(adj_ref[...], t2_ref[...]))


def _logits_kernel(mean_tile_ref, mean_ref, logits_ref, max_ref):
    l = jax.lax.dot_general(mean_tile_ref[...], mean_ref[...],
                            (((1,), (1,)), ((), ())))
    logits_ref[...] = l
    tile_max = jnp.max(l).reshape(1, 1)

    @pl.when(pl.program_id(0) == 0)
    def _():
        max_ref[...] = tile_max

    @pl.when(pl.program_id(0) != 0)
    def _():
        max_ref[...] = jnp.maximum(max_ref[...], tile_max)


def _adj_new_tile(mean_tile, mean_full, max_val, row_base):
    l = jax.lax.dot_general(mean_tile, mean_full, (((1,), (1,)), ((), ())))
    s = jnp.round(l / max_val)
    rows = jax.lax.broadcasted_iota(jnp.int32, s.shape, 0) + row_base
    cols = jax.lax.broadcasted_iota(jnp.int32, s.shape, 1)
    return jnp.where(rows == cols, 1.0, s)


def _degree_kernel(mean_ref, max_ref, d_ref):
    i = pl.program_id(0)
    mean_tile = mean_ref[pl.ds(i * _TM, _TM), :]
    a = _adj_new_tile(mean_tile, mean_ref[...], max_ref[...], i * _TM)
    d_ref[...] = jnp.power(jnp.sum(a, axis=1, keepdims=True), -0.5)


def _h_kernel(mean_ref, max_ref, d_ref, f2_ref, b0_ref, wn1_ref, hw_ref):
    i = pl.program_id(0)
    mean_tile = mean_ref[pl.ds(i * _TM, _TM), :]
    a = _adj_new_tile(mean_tile, mean_ref[...], max_ref[...], i * _TM)
    xd = d_ref[...] * f2_ref[...]
    u = jnp.dot(a, xd)
    h = jax.nn.relu(d_ref[pl.ds(i * _TM, _TM), :] * u + b0_ref[...])
    hw_ref[...] = jnp.dot(h, wn1_ref[...])


def _out_kernel(mean_ref, max_ref, d_ref, hw_ref, b1_ref, out_ref):
    i = pl.program_id(0)
    mean_tile = mean_ref[pl.ds(i * _TM, _TM), :]
    a = _adj_new_tile(mean_tile, mean_ref[...], max_ref[...], i * _TM)
    yd = d_ref[...] * hw_ref[...]
    v = jnp.dot(a, yd)
    out_ref[...] = d_ref[pl.ds(i * _TM, _TM), :] * v + b1_ref[...]


def _whole(shape):
    return pl.BlockSpec(shape, lambda i: (0,) * len(shape))


def kernel(adj, adj_orig, features, W_base, W_mean, W_nc0, b_nc0, W_nc1, b_nc1):
    del adj_orig
    f32 = jnp.float32
    b0 = b_nc0.reshape(1, -1)
    b1 = b_nc1.reshape(1, -1)
    H = W_base.shape[1]
    Z = W_mean.shape[1]
    C = W_nc1.shape[1]

    fb, f2 = pl.pallas_call(
        _feats_kernel,
        out_shape=[jax.ShapeDtypeStruct((_N, H), f32),
                   jax.ShapeDtypeStruct((_N, H), f32)],
    )(features, W_base, W_nc0)

    row_tile = lambda w: pl.BlockSpec((_TM, w), lambda i: (i, 0))

    t2 = pl.pallas_call(
        _hidden_kernel,
        grid=(_NT,),
        in_specs=[row_tile(_N), _whole((_N, H)), _whole((H, Z))],
        out_specs=row_tile(Z),
        out_shape=jax.ShapeDtypeStruct((_N, Z), f32),
    )(adj, fb, W_mean)

    mean = pl.pallas_call(
        _mean_kernel,
        grid=(_NT,),
        in_specs=[row_tile(_N), _whole((_N, Z))],
        out_specs=row_tile(Z),
        out_shape=jax.ShapeDtypeStruct((_N, Z), f32),
    )(adj, t2)

    adj_logits, mx = pl.pallas_call(
        _logits_kernel,
        grid=(_NT,),
        in_specs=[row_tile(Z), _whole((_N, Z))],
        out_specs=[row_tile(_N), pl.BlockSpec((1, 1), lambda i: (0, 0))],
        out_shape=[jax.ShapeDtypeStruct((_N, _N), f32),
                   jax.ShapeDtypeStruct((1, 1), f32)],
    )(mean, mean)

    d = pl.pallas_call(
        _degree_kernel,
        grid=(_NT,),
        in_specs=[_whole((_N, Z)), _whole((1, 1))],
        out_specs=row_tile(1),
        out_shape=jax.ShapeDtypeStruct((_N, 1), f32),
    )(mean, mx)

    hw = pl.pallas_call(
        _h_kernel,
        grid=(_NT,),
        in_specs=[_whole((_N, Z)), _whole((1, 1)), _whole((_N, 1)),
                  _whole((_N, H)), _whole((1, H)), _whole((H, C))],
        out_specs=row_tile(C),
        out_shape=jax.ShapeDtypeStruct((_N, C), f32),
    )(mean, mx, d, f2, b0, W_nc1)

    nc_logits = pl.pallas_call(
        _out_kernel,
        grid=(_NT,),
        in_specs=[_whole((_N, Z)), _whole((1, 1)), _whole((_N, 1)),
                  _whole((_N, C)), _whole((1, C))],
        out_specs=row_tile(C),
        out_shape=jax.ShapeDtypeStruct((_N, C), f32),
    )(mean, mx, d, hw, b1)

    return (nc_logits, adj_logits)


# speedup vs baseline: 6.5521x; 1.1077x over previous
import jax
import jax.numpy as jnp
from jax.experimental import pallas as pl
from jax.experimental.pallas import tpu as pltpu

_N = 4096
_TM = 512
_NT = _N // _TM
_H = 32
_Z = 16
_C = 7


def _adj_new_tile(mean_tile, mean_full, max_val, row_base):
    l = jax.lax.dot_general(mean_tile, mean_full, (((1,), (1,)), ((), ())))
    s = jnp.round(l / max_val)
    rows = jax.lax.broadcasted_iota(jnp.int32, s.shape, 0) + row_base
    cols = jax.lax.broadcasted_iota(jnp.int32, s.shape, 1)
    return jnp.where(rows == cols, 1.0, s)


def _mega_kernel(adj_ref, feats_ref, wb_ref, wm_ref, wn0_ref, b0_ref,
                 wn1_ref, b1_ref,
                 t2_ref, mean_ref, logits_ref, max_ref, d_ref, hw_ref, nc_ref,
                 fb_s, f2_s):
    p = pl.program_id(0)
    i = pl.program_id(1)
    rows = pl.ds(i * _TM, _TM)

    @pl.when((p == 0) & (i == 0))
    def _():
        f = feats_ref[...]
        fb_s[...] = jnp.dot(f, wb_ref[...])
        f2_s[...] = jnp.dot(f, wn0_ref[...])

    @pl.when(p == 0)
    def _():
        t2_ref[rows, :] = jnp.dot(jnp.dot(adj_ref[...], fb_s[...]),
                                  wm_ref[...])

    @pl.when(p == 1)
    def _():
        mean_ref[rows, :] = jax.nn.relu(jnp.dot(adj_ref[...], t2_ref[...]))

    @pl.when(p == 2)
    def _():
        l = jax.lax.dot_general(mean_ref[rows, :], mean_ref[...],
                                (((1,), (1,)), ((), ())))
        logits_ref[...] = l
        tile_max = jnp.max(l).reshape(1, 1)

        @pl.when(i == 0)
        def _():
            max_ref[...] = tile_max

        @pl.when(i != 0)
        def _():
            max_ref[...] = jnp.maximum(max_ref[...], tile_max)

    @pl.when(p == 3)
    def _():
        a = _adj_new_tile(mean_ref[rows, :], mean_ref[...], max_ref[...],
                          i * _TM)
        d_ref[rows, :] = jnp.power(jnp.sum(a, axis=1, keepdims=True), -0.5)

    @pl.when(p == 4)
    def _():
        a = _adj_new_tile(mean_ref[rows, :], mean_ref[...], max_ref[...],
                          i * _TM)
        xd = d_ref[...] * f2_s[...]
        h = jax.nn.relu(d_ref[rows, :] * jnp.dot(a, xd) + b0_ref[...])
        hw_ref[rows, :] = jnp.dot(h, wn1_ref[...])

    @pl.when(p == 5)
    def _():
        a = _adj_new_tile(mean_ref[rows, :], mean_ref[...], max_ref[...],
                          i * _TM)
        yd = d_ref[...] * hw_ref[...]
        nc_ref[rows, :] = d_ref[rows, :] * jnp.dot(a, yd) + b1_ref[...]


def _const(shape):
    return pl.BlockSpec(shape, lambda p, i: (0,) * len(shape))


def kernel(adj, adj_orig, features, W_base, W_mean, W_nc0, b_nc0, W_nc1, b_nc1):
    del adj_orig
    f32 = jnp.float32
    b0 = b_nc0.reshape(1, _H)
    b1 = b_nc1.reshape(1, _C)

    adj_spec = pl.BlockSpec((_TM, _N), lambda p, i: (jnp.where(p <= 1, i, 7), 0))
    logits_spec = pl.BlockSpec(
        (_TM, _N),
        lambda p, i: (jnp.where(p < 2, 0, jnp.where(p == 2, i, 7)), 0))

    t2o, meano, adj_logits, mx, do_, hwo, nco = pl.pallas_call(
        _mega_kernel,
        grid=(6, _NT),
        in_specs=[adj_spec, _const((_N, 128)), _const((128, _H)),
                  _const((_H, _Z)), _const((128, _H)), _const((1, _H)),
                  _const((_H, _C)), _const((1, _C))],
        out_specs=[_const((_N, _Z)), _const((_N, _Z)), logits_spec,
                   _const((1, 1)), _const((_N, 1)), _const((_N, _C)),
                   _const((_N, _C))],
        out_shape=[jax.ShapeDtypeStruct((_N, _Z), f32),
                   jax.ShapeDtypeStruct((_N, _Z), f32),
                   jax.ShapeDtypeStruct((_N, _N), f32),
                   jax.ShapeDtypeStruct((1, 1), f32),
                   jax.ShapeDtypeStruct((_N, 1), f32),
                   jax.ShapeDtypeStruct((_N, _C), f32),
                   jax.ShapeDtypeStruct((_N, _C), f32)],
        scratch_shapes=[pltpu.VMEM((_N, _H), f32), pltpu.VMEM((_N, _H), f32)],
    )(adj, features, W_base, W_mean, W_nc0, b0, W_nc1, b1)

    return (nco, adj_logits)
